# baseline (speedup 1.0000x reference)
import functools

import jax
import jax.numpy as jnp
from jax import lax
from jax.experimental import pallas as pl
from jax.experimental.pallas import tpu as pltpu
from jax.experimental.pallas import tpu_sc as plsc

N_NODES = 10000
N_EDGES = 320000
D = 128

NC = 2
NS = 16
NW = NC * NS
C = 128
CH_W = 80
NB = 16
NBLK = CH_W // NB
NE_PAD = NW * CH_W * C
NP = 10240
RPS = NP // NS


def _sc_agg_body(with_deg, *refs):
    if with_deg:
        (x_hbm, src_hbm, dst_hbm, agg_out, deg_out,
         idx_src_v, idx_dst_v, rows0_v, rows1_v, zrow_v, ones_v, zdeg_v,
         sem0, sem1,
         acc_sh, deg_sh) = refs
    else:
        (x_hbm, src_hbm, dst_hbm, agg_out,
         idx_src_v, idx_dst_v, rows0_v, rows1_v, zrow_v,
         sem0, sem1,
         acc_sh) = refs
    bufs = (rows0_v, rows1_v)
    sems = (sem0, sem1)

    cid = lax.axis_index("c")
    sid = lax.axis_index("s")
    wid = sid * NC + cid

    def zrow_step(i, carry):
        for k in range(D // 16):
            zrow_v[i, pl.ds(k * 16, 16)] = jnp.zeros((16,), jnp.float32)
        return carry
    lax.fori_loop(0, 32, zrow_step, 0)

    if with_deg:
        def zdeg_step(i, carry):
            zdeg_v[pl.ds(i * 16, 16)] = jnp.zeros((16,), jnp.float32)
            return carry
        lax.fori_loop(0, RPS // 16, zdeg_step, 0)
        for k in range(C // 16):
            ones_v[pl.ds(k * 16, 16)] = jnp.ones((16,), jnp.float32)

    r0 = sid * RPS
    for k in range(RPS // 32):
        pltpu.sync_copy(zrow_v, acc_sh.at[pl.ds(r0 + k * 32, 32)])
    if with_deg:
        pltpu.sync_copy(zdeg_v, deg_sh.at[pl.ds(r0, RPS)])

    plsc.subcore_barrier()

    lo = wid * CH_W

    def block_step(b, carry):
        c0 = lo + b * NB
        pltpu.sync_copy(src_hbm.at[pl.ds(c0, NB)], idx_src_v)
        pltpu.sync_copy(dst_hbm.at[pl.ds(c0, NB)], idx_dst_v)
        descs = [None, None]
        descs[0] = pltpu.async_copy(
            x_hbm.at[idx_src_v.at[0]], bufs[0], sems[0])
        for j in range(NB):
            if j + 1 < NB:
                descs[(j + 1) % 2] = pltpu.async_copy(
                    x_hbm.at[idx_src_v.at[j + 1]],
                    bufs[(j + 1) % 2], sems[(j + 1) % 2])
            descs[j % 2].wait()
            pltpu.sync_copy(bufs[j % 2],
                            acc_sh.at[idx_dst_v.at[j]], add=True)
            if with_deg:
                pltpu.sync_copy(ones_v, deg_sh.at[idx_dst_v.at[j]], add=True)
        return carry
    lax.fori_loop(0, NBLK, block_step, 0)

    plsc.subcore_barrier()

    wdescs = [None, None]
    for k in range(RPS // 128):
        pltpu.sync_copy(acc_sh.at[pl.ds(r0 + k * 128, 128)], bufs[k % 2])
        if wdescs[k % 2] is not None:
            wdescs[k % 2].wait()
        wdescs[k % 2] = pltpu.async_copy(
            bufs[k % 2], agg_out.at[cid, pl.ds(r0 + k * 128, 128)],
            sems[k % 2])
    for d in wdescs:
        if d is not None:
            d.wait()
    if with_deg:
        pltpu.sync_copy(deg_sh.at[pl.ds(r0, RPS)], zdeg_v)
        pltpu.sync_copy(zdeg_v, deg_out.at[pl.ds(cid * NP + r0, RPS)])


def _sc_agg(x_tab, src3d, dst3d, with_deg):
    mesh = plsc.VectorSubcoreMesh(core_axis_name="c", subcore_axis_name="s",
                                  num_cores=NC, num_subcores=NS)
    out_type = [jax.ShapeDtypeStruct((NC, NP, D), jnp.float32)]
    scratch = [
        pltpu.VMEM((NB, C), jnp.int32),
        pltpu.VMEM((NB, C), jnp.int32),
        pltpu.VMEM((128, D), jnp.float32),
        pltpu.VMEM((128, D), jnp.float32),
        pltpu.VMEM((32, D), jnp.float32),
    ]
    if with_deg:
        out_type.append(jax.ShapeDtypeStruct((NC * NP,), jnp.float32))
        scratch.append(pltpu.VMEM((C,), jnp.float32))
        scratch.append(pltpu.VMEM((RPS,), jnp.float32))
    scratch.append(pltpu.SemaphoreType.DMA)
    scratch.append(pltpu.SemaphoreType.DMA)
    scratch.append(pltpu.VMEM_SHARED((NP, D), jnp.float32))
    if with_deg:
        scratch.append(pltpu.VMEM_SHARED((NP,), jnp.float32))

    fn = pl.kernel(
        functools.partial(_sc_agg_body, with_deg),
        out_type=tuple(out_type),
        mesh=mesh,
        scratch_types=scratch,
        name="sage_sc_agg",
    )
    return fn(x_tab, src3d, dst3d)


def _tc_right_body(x, wr, b, o):
    dn = (((1,), (0,)), ((), ()))
    o[...] = b[...] + lax.dot_general(x[...], wr[...], dn,
                                      precision=lax.Precision.HIGHEST)


def _tc_right(x_tab, W_r, b, n_out, blk):
    grid = (n_out // blk,)
    return pl.pallas_call(
        _tc_right_body,
        grid=grid,
        in_specs=[
            pl.BlockSpec((blk, D), lambda i: (i, 0)),
            pl.BlockSpec((D, D), lambda i: (0, 0)),
            pl.BlockSpec((1, D), lambda i: (0, 0)),
        ],
        out_specs=pl.BlockSpec((blk, D), lambda i: (i, 0)),
        out_shape=jax.ShapeDtypeStruct((n_out, D), jnp.float32),
        name="sage_tc_right",
    )(x_tab, W_r, b)


def _tc_mean_body(relu, blk, aggp, degp, yr, wl, o):
    agg = aggp[0] + aggp[1]
    ns = blk // 128
    degsum = jnp.maximum(degp[0] + degp[1], 1.0)
    onehot = (lax.broadcasted_iota(jnp.int32, (blk, ns), 0) // 128 ==
              lax.broadcasted_iota(jnp.int32, (blk, ns), 1))
    dn = (((1,), (0,)), ((), ()))
    brows = lax.dot_general(onehot.astype(jnp.float32), degsum, dn,
                            precision=lax.Precision.HIGHEST)
    lmask = (lax.broadcasted_iota(jnp.int32, (blk, 128), 0) % 128 ==
             lax.broadcasted_iota(jnp.int32, (blk, 128), 1))
    deg1 = jnp.sum(jnp.where(lmask, brows, 0.0), axis=1, keepdims=True)
    mean = agg / deg1
    dn = (((1,), (0,)), ((), ()))
    y = lax.dot_general(mean, wl[...], dn,
                        precision=lax.Precision.HIGHEST) + yr[...]
    o[...] = jnp.maximum(y, 0.0) if relu else y


def _tc_mean(aggp, deg2, yr, W_l, relu, n_out, blk):
    grid = (n_out // blk,)
    return pl.pallas_call(
        functools.partial(_tc_mean_body, relu, blk),
        grid=grid,
        in_specs=[
            pl.BlockSpec((NC, blk, D), lambda i: (0, i, 0)),
            pl.BlockSpec((NC, blk // 128, 128), lambda i: (0, i, 0)),
            pl.BlockSpec((blk, D), lambda i: (i, 0)),
            pl.BlockSpec((D, D), lambda i: (0, 0)),
        ],
        out_specs=pl.BlockSpec((blk, D), lambda i: (i, 0)),
        out_shape=jax.ShapeDtypeStruct((n_out, D), jnp.float32),
        name="sage_tc_mean",
    )(aggp, deg2, yr, W_l)


def kernel(x, edge_index, W1_l, b1, W1_r, W2_l, b2, W2_r):
    npad = NE_PAD - N_EDGES
    cyc = jnp.arange(npad, dtype=jnp.int32) % (NP - N_NODES)
    src2d = jnp.concatenate(
        [edge_index[0].astype(jnp.int32), cyc]).reshape(NW * CH_W, C)
    dst2d = jnp.concatenate(
        [edge_index[1].astype(jnp.int32), N_NODES + cyc]).reshape(NW * CH_W, C)
    b1r = b1.reshape(1, D)
    b2r = b2.reshape(1, D)

    aggx, deg_flat = _sc_agg(x, src2d, dst2d, with_deg=True)
    yr1 = _tc_right(x, W1_r, b1r, n_out=NP, blk=1024)
    deg2 = deg_flat.reshape(NC, NP // 128, 128)
    h = _tc_mean(aggx, deg2, yr1, W1_l, relu=True, n_out=NP, blk=1024)
    aggh, = _sc_agg(h, src2d, dst2d, with_deg=False)
    yr2 = _tc_right(h, W2_r, b2r, n_out=NP, blk=1024)
    out = _tc_mean(aggh, deg2, yr2, W2_l, relu=False, n_out=NP, blk=1024)
    return out[:N_NODES]


# speedup vs baseline: 1.0582x; 1.0582x over previous
import functools

import jax
import jax.numpy as jnp
from jax import lax
from jax.experimental import pallas as pl
from jax.experimental.pallas import tpu as pltpu
from jax.experimental.pallas import tpu_sc as plsc

N_NODES = 10000
N_EDGES = 320000
D = 128

NC = 2
NS = 16
NW = NC * NS
C = 128
CH_W = 80
NB = 16
NBLK = CH_W // NB
NE_PAD = NW * CH_W * C
NP = 10240
RPS = NP // NS


def _sc_agg_body(with_deg, *refs):
    if with_deg:
        (x_hbm, src_hbm, dst_hbm, agg_out, deg_out,
         idx_src_v, idx_dst_v, rows0_v, rows1_v, zrow_v, ones_v, zdeg_v,
         sem0, sem1,
         acc_sh, deg_sh) = refs
    else:
        (x_hbm, src_hbm, dst_hbm, agg_out,
         idx_src_v, idx_dst_v, rows0_v, rows1_v, zrow_v,
         sem0, sem1,
         acc_sh) = refs
    bufs = (rows0_v, rows1_v)
    sems = (sem0, sem1)

    cid = lax.axis_index("c")
    sid = lax.axis_index("s")
    wid = sid * NC + cid

    def zrow_step(i, carry):
        for k in range(D // 16):
            zrow_v[i, pl.ds(k * 16, 16)] = jnp.zeros((16,), jnp.float32)
        return carry
    lax.fori_loop(0, 32, zrow_step, 0)

    if with_deg:
        def zdeg_step(i, carry):
            zdeg_v[pl.ds(i * 16, 16)] = jnp.zeros((16,), jnp.float32)
            return carry
        lax.fori_loop(0, RPS // 16, zdeg_step, 0)
        for k in range(C // 16):
            ones_v[pl.ds(k * 16, 16)] = jnp.ones((16,), jnp.float32)

    r0 = sid * RPS
    for k in range(RPS // 32):
        pltpu.sync_copy(zrow_v, acc_sh.at[pl.ds(r0 + k * 32, 32)])
    if with_deg:
        pltpu.sync_copy(zdeg_v, deg_sh.at[pl.ds(r0, RPS)])

    plsc.subcore_barrier()

    lo = wid * CH_W

    def block_step(b, carry):
        c0 = lo + b * NB
        pltpu.sync_copy(src_hbm.at[pl.ds(c0, NB)], idx_src_v)
        pltpu.sync_copy(dst_hbm.at[pl.ds(c0, NB)], idx_dst_v)
        descs = [None, None]
        descs[0] = pltpu.async_copy(
            x_hbm.at[idx_src_v.at[0]], bufs[0], sems[0])
        for j in range(NB):
            if j + 1 < NB:
                descs[(j + 1) % 2] = pltpu.async_copy(
                    x_hbm.at[idx_src_v.at[j + 1]],
                    bufs[(j + 1) % 2], sems[(j + 1) % 2])
            descs[j % 2].wait()
            pltpu.sync_copy(bufs[j % 2],
                            acc_sh.at[idx_dst_v.at[j]], add=True)
            if with_deg:
                pltpu.sync_copy(ones_v, deg_sh.at[idx_dst_v.at[j]], add=True)
        return carry
    lax.fori_loop(0, NBLK, block_step, 0)

    plsc.subcore_barrier()

    wdescs = [None, None]
    for k in range(RPS // 128):
        pltpu.sync_copy(acc_sh.at[pl.ds(r0 + k * 128, 128)], bufs[k % 2])
        if wdescs[k % 2] is not None:
            wdescs[k % 2].wait()
        wdescs[k % 2] = pltpu.async_copy(
            bufs[k % 2], agg_out.at[cid, pl.ds(r0 + k * 128, 128)],
            sems[k % 2])
    for d in wdescs:
        if d is not None:
            d.wait()
    if with_deg:
        pltpu.sync_copy(deg_sh.at[pl.ds(r0, RPS)], zdeg_v)
        pltpu.sync_copy(zdeg_v, deg_out.at[pl.ds(cid * NP + r0, RPS)])


def _sc_agg(x_tab, src3d, dst3d, with_deg):
    mesh = plsc.VectorSubcoreMesh(core_axis_name="c", subcore_axis_name="s",
                                  num_cores=NC, num_subcores=NS)
    out_type = [jax.ShapeDtypeStruct((NC, NP, D), jnp.float32)]
    scratch = [
        pltpu.VMEM((NB, C), jnp.int32),
        pltpu.VMEM((NB, C), jnp.int32),
        pltpu.VMEM((128, D), jnp.float32),
        pltpu.VMEM((128, D), jnp.float32),
        pltpu.VMEM((32, D), jnp.float32),
    ]
    if with_deg:
        out_type.append(jax.ShapeDtypeStruct((NC * NP,), jnp.float32))
        scratch.append(pltpu.VMEM((C,), jnp.float32))
        scratch.append(pltpu.VMEM((RPS,), jnp.float32))
    scratch.append(pltpu.SemaphoreType.DMA)
    scratch.append(pltpu.SemaphoreType.DMA)
    scratch.append(pltpu.VMEM_SHARED((NP, D), jnp.float32))
    if with_deg:
        scratch.append(pltpu.VMEM_SHARED((NP,), jnp.float32))

    fn = pl.kernel(
        functools.partial(_sc_agg_body, with_deg),
        out_type=tuple(out_type),
        mesh=mesh,
        scratch_types=scratch,
        name="sage_sc_agg",
    )
    return fn(x_tab, src3d, dst3d)


def _tc_right_body(x, wr, b, o):
    dn = (((1,), (0,)), ((), ()))
    o[...] = b[...] + lax.dot_general(x[...], wr[...], dn,
                                      precision=lax.Precision.HIGHEST)


def _tc_right(x_tab, W_r, b, n_out, blk):
    grid = (n_out // blk,)
    return pl.pallas_call(
        _tc_right_body,
        grid=grid,
        in_specs=[
            pl.BlockSpec((blk, D), lambda i: (i, 0)),
            pl.BlockSpec((D, D), lambda i: (0, 0)),
            pl.BlockSpec((1, D), lambda i: (0, 0)),
        ],
        out_specs=pl.BlockSpec((blk, D), lambda i: (i, 0)),
        out_shape=jax.ShapeDtypeStruct((n_out, D), jnp.float32),
        name="sage_tc_right",
    )(x_tab, W_r, b)


def _tc_mean_body(relu, blk, aggp, degp, yr, wl, o):
    agg = aggp[0] + aggp[1]
    ns = blk // 128
    degsum = jnp.maximum(degp[0] + degp[1], 1.0)
    onehot = (lax.broadcasted_iota(jnp.int32, (blk, ns), 0) // 128 ==
              lax.broadcasted_iota(jnp.int32, (blk, ns), 1))
    dn = (((1,), (0,)), ((), ()))
    brows = lax.dot_general(onehot.astype(jnp.float32), degsum, dn)
    lmask = (lax.broadcasted_iota(jnp.int32, (blk, 128), 0) % 128 ==
             lax.broadcasted_iota(jnp.int32, (blk, 128), 1))
    deg1 = jnp.sum(jnp.where(lmask, brows, 0.0), axis=1, keepdims=True)
    mean = agg / deg1
    dn = (((1,), (0,)), ((), ()))
    y = lax.dot_general(mean, wl[...], dn,
                        precision=lax.Precision.HIGHEST) + yr[...]
    o[...] = jnp.maximum(y, 0.0) if relu else y


def _tc_mean(aggp, deg2, yr, W_l, relu, n_out, blk):
    grid = ((n_out + blk - 1) // blk,)
    return pl.pallas_call(
        functools.partial(_tc_mean_body, relu, blk),
        grid=grid,
        in_specs=[
            pl.BlockSpec((NC, blk, D), lambda i: (0, i, 0)),
            pl.BlockSpec((NC, blk // 128, 128), lambda i: (0, i, 0)),
            pl.BlockSpec((blk, D), lambda i: (i, 0)),
            pl.BlockSpec((D, D), lambda i: (0, 0)),
        ],
        out_specs=pl.BlockSpec((blk, D), lambda i: (i, 0)),
        out_shape=jax.ShapeDtypeStruct((n_out, D), jnp.float32),
        name="sage_tc_mean",
    )(aggp, deg2, yr, W_l)


def kernel(x, edge_index, W1_l, b1, W1_r, W2_l, b2, W2_r):
    npad = NE_PAD - N_EDGES
    cyc = jnp.arange(npad, dtype=jnp.int32) % (NP - N_NODES)
    src2d = jnp.concatenate(
        [edge_index[0].astype(jnp.int32), cyc]).reshape(NW * CH_W, C)
    dst2d = jnp.concatenate(
        [edge_index[1].astype(jnp.int32), N_NODES + cyc]).reshape(NW * CH_W, C)
    b1r = b1.reshape(1, D)
    b2r = b2.reshape(1, D)

    aggx, deg_flat = _sc_agg(x, src2d, dst2d, with_deg=True)
    yr1 = _tc_right(x, W1_r, b1r, n_out=NP, blk=1024)
    deg2 = deg_flat.reshape(NC, NP // 128, 128)
    h = _tc_mean(aggx, deg2, yr1, W1_l, relu=True, n_out=NP, blk=1024)
    aggh, = _sc_agg(h, src2d, dst2d, with_deg=False)
    yr2 = _tc_right(h, W2_r, b2r, n_out=NP, blk=1024)
    out = _tc_mean(aggh, deg2, yr2, W2_l, relu=False,
                   n_out=N_NODES, blk=1024)
    return out


# speedup vs baseline: 1.0614x; 1.0030x over previous
import functools

import jax
import jax.numpy as jnp
from jax import lax
from jax.experimental import pallas as pl
from jax.experimental.pallas import tpu as pltpu
from jax.experimental.pallas import tpu_sc as plsc

N_NODES = 10000
N_EDGES = 320000
D = 128

NC = 2
NS = 16
NW = NC * NS
C = 128
CH_W = 80
NB = 16
NBLK = CH_W // NB
NE_PAD = NW * CH_W * C
NP = 10240
RPS = NP // NS


def _sc_agg_body(with_deg, *refs):
    if with_deg:
        (x_hbm, src_hbm, dst_hbm, agg_out, deg_out,
         idx_src_v, idx_dst_v, rows0_v, rows1_v, zrow_v, ones_v, zdeg_v,
         gs0, gs1, ss0, ss1, ds0, ds1,
         acc_sh, deg_sh) = refs
        dsems = (ds0, ds1)
    else:
        (x_hbm, src_hbm, dst_hbm, agg_out,
         idx_src_v, idx_dst_v, rows0_v, rows1_v, zrow_v,
         gs0, gs1, ss0, ss1, ds0, ds1,
         acc_sh) = refs
    bufs = (rows0_v, rows1_v)
    sems = (gs0, gs1)
    ssems = (ss0, ss1)

    cid = lax.axis_index("c")
    sid = lax.axis_index("s")
    wid = sid * NC + cid

    def zrow_step(i, carry):
        for k in range(D // 16):
            zrow_v[i, pl.ds(k * 16, 16)] = jnp.zeros((16,), jnp.float32)
        return carry
    lax.fori_loop(0, 32, zrow_step, 0)

    if with_deg:
        def zdeg_step(i, carry):
            zdeg_v[pl.ds(i * 16, 16)] = jnp.zeros((16,), jnp.float32)
            return carry
        lax.fori_loop(0, RPS // 16, zdeg_step, 0)
        for k in range(C // 16):
            ones_v[pl.ds(k * 16, 16)] = jnp.ones((16,), jnp.float32)

    r0 = sid * RPS
    for k in range(RPS // 32):
        pltpu.sync_copy(zrow_v, acc_sh.at[pl.ds(r0 + k * 32, 32)])
    if with_deg:
        pltpu.sync_copy(zdeg_v, deg_sh.at[pl.ds(r0, RPS)])

    plsc.subcore_barrier()

    lo = wid * CH_W

    def block_step(b, carry):
        c0 = lo + b * NB
        pltpu.sync_copy(src_hbm.at[pl.ds(c0, NB)], idx_src_v)
        pltpu.sync_copy(dst_hbm.at[pl.ds(c0, NB)], idx_dst_v)
        gd = [None, None]
        sd = [None, None]
        dd = [None, None]
        gd[0] = pltpu.async_copy(x_hbm.at[idx_src_v.at[0]], bufs[0], sems[0])
        for j in range(NB):
            if j + 1 < NB:
                if sd[(j + 1) % 2] is not None:
                    sd[(j + 1) % 2].wait()
                gd[(j + 1) % 2] = pltpu.async_copy(
                    x_hbm.at[idx_src_v.at[j + 1]],
                    bufs[(j + 1) % 2], sems[(j + 1) % 2])
            gd[j % 2].wait()
            sd[j % 2] = pltpu.async_copy(
                bufs[j % 2], acc_sh.at[idx_dst_v.at[j]], ssems[j % 2],
                add=True)
            if with_deg:
                if dd[j % 2] is not None:
                    dd[j % 2].wait()
                dd[j % 2] = pltpu.async_copy(
                    ones_v, deg_sh.at[idx_dst_v.at[j]], dsems[j % 2],
                    add=True)
        for d in sd + (dd if with_deg else []):
            if d is not None:
                d.wait()
        return carry
    lax.fori_loop(0, NBLK, block_step, 0)

    plsc.subcore_barrier()

    wdescs = [None, None]
    for k in range(RPS // 128):
        pltpu.sync_copy(acc_sh.at[pl.ds(r0 + k * 128, 128)], bufs[k % 2])
        if wdescs[k % 2] is not None:
            wdescs[k % 2].wait()
        wdescs[k % 2] = pltpu.async_copy(
            bufs[k % 2], agg_out.at[cid, pl.ds(r0 + k * 128, 128)],
            sems[k % 2])
    for d in wdescs:
        if d is not None:
            d.wait()
    if with_deg:
        pltpu.sync_copy(deg_sh.at[pl.ds(r0, RPS)], zdeg_v)
        pltpu.sync_copy(zdeg_v, deg_out.at[pl.ds(cid * NP + r0, RPS)])


def _sc_agg(x_tab, src3d, dst3d, with_deg):
    mesh = plsc.VectorSubcoreMesh(core_axis_name="c", subcore_axis_name="s",
                                  num_cores=NC, num_subcores=NS)
    out_type = [jax.ShapeDtypeStruct((NC, NP, D), jnp.float32)]
    scratch = [
        pltpu.VMEM((NB, C), jnp.int32),
        pltpu.VMEM((NB, C), jnp.int32),
        pltpu.VMEM((128, D), jnp.float32),
        pltpu.VMEM((128, D), jnp.float32),
        pltpu.VMEM((32, D), jnp.float32),
    ]
    if with_deg:
        out_type.append(jax.ShapeDtypeStruct((NC * NP,), jnp.float32))
        scratch.append(pltpu.VMEM((C,), jnp.float32))
        scratch.append(pltpu.VMEM((RPS,), jnp.float32))
    for _ in range(6):
        scratch.append(pltpu.SemaphoreType.DMA)
    scratch.append(pltpu.VMEM_SHARED((NP, D), jnp.float32))
    if with_deg:
        scratch.append(pltpu.VMEM_SHARED((NP,), jnp.float32))

    fn = pl.kernel(
        functools.partial(_sc_agg_body, with_deg),
        out_type=tuple(out_type),
        mesh=mesh,
        scratch_types=scratch,
        name="sage_sc_agg",
    )
    return fn(x_tab, src3d, dst3d)


def _tc_right_body(x, wr, b, o):
    dn = (((1,), (0,)), ((), ()))
    o[...] = b[...] + lax.dot_general(x[...], wr[...], dn,
                                      precision=lax.Precision.HIGHEST)


def _tc_right(x_tab, W_r, b, n_out, blk):
    grid = (n_out // blk,)
    return pl.pallas_call(
        _tc_right_body,
        grid=grid,
        in_specs=[
            pl.BlockSpec((blk, D), lambda i: (i, 0)),
            pl.BlockSpec((D, D), lambda i: (0, 0)),
            pl.BlockSpec((1, D), lambda i: (0, 0)),
        ],
        out_specs=pl.BlockSpec((blk, D), lambda i: (i, 0)),
        out_shape=jax.ShapeDtypeStruct((n_out, D), jnp.float32),
        name="sage_tc_right",
    )(x_tab, W_r, b)


def _tc_mean_body(relu, blk, aggp, degp, yr, wl, o):
    agg = aggp[0] + aggp[1]
    ns = blk // 128
    degsum = jnp.maximum(degp[0] + degp[1], 1.0)
    onehot = (lax.broadcasted_iota(jnp.int32, (blk, ns), 0) // 128 ==
              lax.broadcasted_iota(jnp.int32, (blk, ns), 1))
    dn = (((1,), (0,)), ((), ()))
    brows = lax.dot_general(onehot.astype(jnp.float32), degsum, dn)
    lmask = (lax.broadcasted_iota(jnp.int32, (blk, 128), 0) % 128 ==
             lax.broadcasted_iota(jnp.int32, (blk, 128), 1))
    deg1 = jnp.sum(jnp.where(lmask, brows, 0.0), axis=1, keepdims=True)
    mean = agg / deg1
    dn = (((1,), (0,)), ((), ()))
    y = lax.dot_general(mean, wl[...], dn,
                        precision=lax.Precision.HIGHEST) + yr[...]
    o[...] = jnp.maximum(y, 0.0) if relu else y


def _tc_mean(aggp, deg2, yr, W_l, relu, n_out, blk):
    grid = ((n_out + blk - 1) // blk,)
    return pl.pallas_call(
        functools.partial(_tc_mean_body, relu, blk),
        grid=grid,
        in_specs=[
            pl.BlockSpec((NC, blk, D), lambda i: (0, i, 0)),
            pl.BlockSpec((NC, blk // 128, 128), lambda i: (0, i, 0)),
            pl.BlockSpec((blk, D), lambda i: (i, 0)),
            pl.BlockSpec((D, D), lambda i: (0, 0)),
        ],
        out_specs=pl.BlockSpec((blk, D), lambda i: (i, 0)),
        out_shape=jax.ShapeDtypeStruct((n_out, D), jnp.float32),
        name="sage_tc_mean",
    )(aggp, deg2, yr, W_l)


def kernel(x, edge_index, W1_l, b1, W1_r, W2_l, b2, W2_r):
    npad = NE_PAD - N_EDGES
    cyc = jnp.arange(npad, dtype=jnp.int32) % (NP - N_NODES)
    src2d = jnp.concatenate(
        [edge_index[0].astype(jnp.int32), cyc]).reshape(NW * CH_W, C)
    dst2d = jnp.concatenate(
        [edge_index[1].astype(jnp.int32), N_NODES + cyc]).reshape(NW * CH_W, C)
    b1r = b1.reshape(1, D)
    b2r = b2.reshape(1, D)

    aggx, deg_flat = _sc_agg(x, src2d, dst2d, with_deg=True)
    yr1 = _tc_right(x, W1_r, b1r, n_out=NP, blk=1024)
    deg2 = deg_flat.reshape(NC, NP // 128, 128)
    h = _tc_mean(aggx, deg2, yr1, W1_l, relu=True, n_out=NP, blk=1024)
    aggh, = _sc_agg(h, src2d, dst2d, with_deg=False)
    yr2 = _tc_right(h, W2_r, b2r, n_out=NP, blk=1024)
    out = _tc_mean(aggh, deg2, yr2, W2_l, relu=False,
                   n_out=N_NODES, blk=1024)
    return out


# speedup vs baseline: 1.0688x; 1.0070x over previous
import functools

import jax
import jax.numpy as jnp
from jax import lax
from jax.experimental import pallas as pl
from jax.experimental.pallas import tpu as pltpu
from jax.experimental.pallas import tpu_sc as plsc

N_NODES = 10000
N_EDGES = 320000
D = 128

NC = 2
NS = 16
NW = NC * NS
C = 128
CH_W = 80
NB = 16
NBLK = CH_W // NB
NE_PAD = NW * CH_W * C
NP = 10240
RPS = NP // NS


def _sc_agg_body(with_deg, *refs):
    if with_deg:
        (x_hbm, src_hbm, dst_hbm, agg_out, deg_out,
         idx_src_v, idx_dst_v, rows0_v, rows1_v, zrow_v, ones_v, zdeg_v,
         gs0, gs1, ss0, ss1, ds0, ds1,
         acc_sh, deg_sh) = refs
        dsems = (ds0, ds1)
    else:
        (x_hbm, src_hbm, dst_hbm, agg_out,
         idx_src_v, idx_dst_v, rows0_v, rows1_v, zrow_v,
         gs0, gs1, ss0, ss1, ds0, ds1,
         acc_sh) = refs
    bufs = (rows0_v, rows1_v)
    sems = (gs0, gs1)
    ssems = (ss0, ss1)

    cid = lax.axis_index("c")
    sid = lax.axis_index("s")
    wid = sid * NC + cid

    def zrow_step(i, carry):
        for k in range(D // 16):
            zrow_v[i, pl.ds(k * 16, 16)] = jnp.zeros((16,), jnp.float32)
        return carry
    lax.fori_loop(0, 32, zrow_step, 0)

    if with_deg:
        def zdeg_step(i, carry):
            zdeg_v[pl.ds(i * 16, 16)] = jnp.zeros((16,), jnp.float32)
            return carry
        lax.fori_loop(0, RPS // 16, zdeg_step, 0)
        for k in range(C // 16):
            ones_v[pl.ds(k * 16, 16)] = jnp.ones((16,), jnp.float32)

    r0 = sid * RPS
    zd = []
    for k in range(RPS // 32):
        zd.append(pltpu.async_copy(zrow_v, acc_sh.at[pl.ds(r0 + k * 32, 32)],
                                   gs0))
    if with_deg:
        zd.append(pltpu.async_copy(zdeg_v, deg_sh.at[pl.ds(r0, RPS)], gs0))

    for d in zd:
        d.wait()
    plsc.subcore_barrier()

    lo = wid * CH_W

    def block_step(b, carry):
        c0 = lo + b * NB
        pltpu.sync_copy(src_hbm.at[pl.ds(c0, NB)], idx_src_v)
        pltpu.sync_copy(dst_hbm.at[pl.ds(c0, NB)], idx_dst_v)
        gd = [None, None]
        sd = [None, None]
        dd = [None, None]
        gd[0] = pltpu.async_copy(x_hbm.at[idx_src_v.at[0]], bufs[0], sems[0])
        for j in range(NB):
            if j + 1 < NB:
                if sd[(j + 1) % 2] is not None:
                    sd[(j + 1) % 2].wait()
                gd[(j + 1) % 2] = pltpu.async_copy(
                    x_hbm.at[idx_src_v.at[j + 1]],
                    bufs[(j + 1) % 2], sems[(j + 1) % 2])
            gd[j % 2].wait()
            sd[j % 2] = pltpu.async_copy(
                bufs[j % 2], acc_sh.at[idx_dst_v.at[j]], ssems[j % 2],
                add=True)
            if with_deg:
                if dd[j % 2] is not None:
                    dd[j % 2].wait()
                dd[j % 2] = pltpu.async_copy(
                    ones_v, deg_sh.at[idx_dst_v.at[j]], dsems[j % 2],
                    add=True)
        for d in sd + (dd if with_deg else []):
            if d is not None:
                d.wait()
        return carry
    lax.fori_loop(0, NBLK, block_step, 0)

    plsc.subcore_barrier()

    wdescs = [None, None]
    for k in range(RPS // 128):
        pltpu.sync_copy(acc_sh.at[pl.ds(r0 + k * 128, 128)], bufs[k % 2])
        if wdescs[k % 2] is not None:
            wdescs[k % 2].wait()
        wdescs[k % 2] = pltpu.async_copy(
            bufs[k % 2], agg_out.at[cid, pl.ds(r0 + k * 128, 128)],
            sems[k % 2])
    for d in wdescs:
        if d is not None:
            d.wait()
    if with_deg:
        pltpu.sync_copy(deg_sh.at[pl.ds(r0, RPS)], zdeg_v)
        pltpu.sync_copy(zdeg_v, deg_out.at[pl.ds(cid * NP + r0, RPS)])


def _sc_agg(x_tab, src3d, dst3d, with_deg):
    mesh = plsc.VectorSubcoreMesh(core_axis_name="c", subcore_axis_name="s",
                                  num_cores=NC, num_subcores=NS)
    out_type = [jax.ShapeDtypeStruct((NC, NP, D), jnp.float32)]
    scratch = [
        pltpu.VMEM((NB, C), jnp.int32),
        pltpu.VMEM((NB, C), jnp.int32),
        pltpu.VMEM((128, D), jnp.float32),
        pltpu.VMEM((128, D), jnp.float32),
        pltpu.VMEM((32, D), jnp.float32),
    ]
    if with_deg:
        out_type.append(jax.ShapeDtypeStruct((NC * NP,), jnp.float32))
        scratch.append(pltpu.VMEM((C,), jnp.float32))
        scratch.append(pltpu.VMEM((RPS,), jnp.float32))
    for _ in range(6):
        scratch.append(pltpu.SemaphoreType.DMA)
    scratch.append(pltpu.VMEM_SHARED((NP, D), jnp.float32))
    if with_deg:
        scratch.append(pltpu.VMEM_SHARED((NP,), jnp.float32))

    fn = pl.kernel(
        functools.partial(_sc_agg_body, with_deg),
        out_type=tuple(out_type),
        mesh=mesh,
        scratch_types=scratch,
        name="sage_sc_agg",
    )
    return fn(x_tab, src3d, dst3d)


def _tc_right_body(x, wr, b, o):
    dn = (((1,), (0,)), ((), ()))
    o[...] = b[...] + lax.dot_general(x[...], wr[...], dn,
                                      precision=lax.Precision.HIGHEST)


def _tc_right(x_tab, W_r, b, n_out, blk):
    grid = (n_out // blk,)
    return pl.pallas_call(
        _tc_right_body,
        grid=grid,
        in_specs=[
            pl.BlockSpec((blk, D), lambda i: (i, 0)),
            pl.BlockSpec((D, D), lambda i: (0, 0)),
            pl.BlockSpec((1, D), lambda i: (0, 0)),
        ],
        out_specs=pl.BlockSpec((blk, D), lambda i: (i, 0)),
        out_shape=jax.ShapeDtypeStruct((n_out, D), jnp.float32),
        name="sage_tc_right",
    )(x_tab, W_r, b)


def _tc_mean_body(relu, blk, aggp, degp, yr, wl, o):
    agg = aggp[0] + aggp[1]
    ns = blk // 128
    degsum = jnp.maximum(degp[0] + degp[1], 1.0)
    onehot = (lax.broadcasted_iota(jnp.int32, (blk, ns), 0) // 128 ==
              lax.broadcasted_iota(jnp.int32, (blk, ns), 1))
    dn = (((1,), (0,)), ((), ()))
    brows = lax.dot_general(onehot.astype(jnp.float32), degsum, dn)
    lmask = (lax.broadcasted_iota(jnp.int32, (blk, 128), 0) % 128 ==
             lax.broadcasted_iota(jnp.int32, (blk, 128), 1))
    deg1 = jnp.sum(jnp.where(lmask, brows, 0.0), axis=1, keepdims=True)
    mean = agg / deg1
    dn = (((1,), (0,)), ((), ()))
    y = lax.dot_general(mean, wl[...], dn,
                        precision=lax.Precision.HIGHEST) + yr[...]
    o[...] = jnp.maximum(y, 0.0) if relu else y


def _tc_mean(aggp, deg2, yr, W_l, relu, n_out, blk):
    grid = ((n_out + blk - 1) // blk,)
    return pl.pallas_call(
        functools.partial(_tc_mean_body, relu, blk),
        grid=grid,
        in_specs=[
            pl.BlockSpec((NC, blk, D), lambda i: (0, i, 0)),
            pl.BlockSpec((NC, blk // 128, 128), lambda i: (0, i, 0)),
            pl.BlockSpec((blk, D), lambda i: (i, 0)),
            pl.BlockSpec((D, D), lambda i: (0, 0)),
        ],
        out_specs=pl.BlockSpec((blk, D), lambda i: (i, 0)),
        out_shape=jax.ShapeDtypeStruct((n_out, D), jnp.float32),
        name="sage_tc_mean",
    )(aggp, deg2, yr, W_l)


def kernel(x, edge_index, W1_l, b1, W1_r, W2_l, b2, W2_r):
    npad = NE_PAD - N_EDGES
    cyc = jnp.arange(npad, dtype=jnp.int32) % (NP - N_NODES)
    src2d = jnp.concatenate(
        [edge_index[0].astype(jnp.int32), cyc]).reshape(NW * CH_W, C)
    dst2d = jnp.concatenate(
        [edge_index[1].astype(jnp.int32), N_NODES + cyc]).reshape(NW * CH_W, C)
    b1r = b1.reshape(1, D)
    b2r = b2.reshape(1, D)

    aggx, deg_flat = _sc_agg(x, src2d, dst2d, with_deg=True)
    yr1 = _tc_right(x, W1_r, b1r, n_out=NP, blk=1024)
    deg2 = deg_flat.reshape(NC, NP // 128, 128)
    h = _tc_mean(aggx, deg2, yr1, W1_l, relu=True, n_out=NP, blk=1024)
    aggh, = _sc_agg(h, src2d, dst2d, with_deg=False)
    yr2 = _tc_right(h, W2_r, b2r, n_out=NP, blk=1024)
    out = _tc_mean(aggh, deg2, yr2, W2_l, relu=False,
                   n_out=N_NODES, blk=1024)
    return out
